# baseline (speedup 1.0000x reference)
import functools

import numpy as np
import jax
import jax.numpy as jnp
from jax import lax
from jax.experimental import pallas as pl
from jax.experimental.pallas import tpu as pltpu
from jax.experimental.pallas import tpu_sc as plsc

NUM_PAIRS = 4096
IRREPS_DIM = 128
NUM_IRREPS = 64
NUM_SCALAR = 32
N_EDGES = 320000

_SHIFT_OFF = NUM_IRREPS

_INFO = plsc.get_sparse_core_info()
_NC, _NS, _L = _INFO.num_cores, _INFO.num_subcores, _INFO.num_lanes
_NW = _NC * _NS
_PER_W = N_EDGES // _NW
_B = 80
_CHUNKS = _PER_W // _B
_NVREG = IRREPS_DIM // _L


def _sc_body(feat_hbm, etype_hbm, table_hbm, out_hbm,
             idx_v, x_v, t_v, tab_sh,
             sem_tab, sem_idx, sem_t0, sem_t1, sem_x0, sem_x1,
             sem_o0, sem_o1):
    sid = lax.axis_index("s")
    wid = sid * _NC + lax.axis_index("c")
    w_base = wid * _PER_W

    @pl.when(sid == 0)
    def _():
        pltpu.async_copy(table_hbm, tab_sh, sem_tab).wait()

    pltpu.async_copy(etype_hbm.at[pl.ds(w_base, _PER_W)], idx_v,
                     sem_idx).wait()
    plsc.subcore_barrier()

    lane = lax.iota(jnp.int32, _L)
    lidx = []
    for j in range(2, _NVREG):
        col = ((16 * j - 32 + lane) * 10923) >> 15
        lidx.append(col - (0 if j <= 4 else 16))
    dnums = lax.GatherDimensionNumbers(
        offset_dims=(), collapsed_slice_dims=(0,), start_index_map=(0,))

    def perm(v, i):
        return lax.gather(v, i[:, None], dnums, (1,),
                          mode=lax.GatherScatterMode.PROMISE_IN_BOUNDS)

    sems_t = (sem_t0, sem_t1)
    sems_x = (sem_x0, sem_x1)
    sems_o = (sem_o0, sem_o1)

    def _in_args(g, slot):
        base = w_base + g * _B
        return ((tab_sh.at[idx_v.at[pl.ds(g * _B, _B)]], t_v.at[slot],
                 sems_t[slot]),
                (feat_hbm.at[pl.ds(base, _B)], x_v.at[slot], sems_x[slot]))

    def issue_in(g, slot):
        for a in _in_args(g, slot):
            pltpu.async_copy(*a)

    def wait_in(g, slot):
        for a in _in_args(g, slot):
            pltpu.make_async_copy(*a).wait()

    def _out_args(g, slot):
        base = w_base + g * _B
        return (x_v.at[slot], out_hbm.at[pl.ds(base, _B)], sems_o[slot])

    def issue_out(g, slot):
        pltpu.async_copy(*_out_args(g, slot))

    def wait_out(g, slot):
        pltpu.make_async_copy(*_out_args(g, slot)).wait()

    def compute(slot):
        def edge(e, c2):
            for j in range(2):
                sl = pl.ds(j * _L, _L)
                hsl = pl.ds(_SHIFT_OFF + j * _L, _L)
                x_v[slot, e, sl] = (x_v[slot, e, sl] * t_v[slot, e, sl]
                                    + t_v[slot, e, hsl])
            t2 = t_v[slot, e, pl.ds(2 * _L, _L)]
            t3 = t_v[slot, e, pl.ds(3 * _L, _L)]
            for j in range(2, _NVREG):
                sl = pl.ds(j * _L, _L)
                s = perm(t2 if j <= 4 else t3, lidx[j - 2])
                x_v[slot, e, sl] = x_v[slot, e, sl] * s
            return c2

        lax.fori_loop(0, _B, edge, 0, unroll=4)

    def step(g, slot):
        wait_in(g, slot)
        compute(slot)
        issue_out(g, slot)
        wait_out(g - 1, 1 - slot)
        issue_in(g + 1, 1 - slot)

    issue_in(0, 0)
    issue_in(1, 1)

    wait_in(0, 0)
    compute(0)
    issue_out(0, 0)

    def main(gg, carry):
        for b in (0, 1):
            step(2 * gg + 1 + b, 1 - b)
        return carry

    _M = (_CHUNKS - 3) // 2
    lax.fori_loop(0, _M, main, 0, unroll=False)

    for g in range(2 * _M + 1, _CHUNKS - 1):
        step(g, g % 2)
    g = _CHUNKS - 1
    wait_in(g, g % 2)
    compute(g % 2)
    issue_out(g, g % 2)
    wait_out(g - 1, 1 - g % 2)
    wait_out(g, g % 2)


@functools.partial(jax.jit, static_argnames=())
def _run(edge_features, etype_flat, table_c):
    mesh = plsc.VectorSubcoreMesh(core_axis_name="c", subcore_axis_name="s")
    call = pl.kernel(
        _sc_body,
        mesh=mesh,
        out_type=jax.ShapeDtypeStruct((N_EDGES, IRREPS_DIM), jnp.float32),
        scratch_types=[
            pltpu.VMEM((_PER_W,), jnp.int32),
            pltpu.VMEM((2, _B, IRREPS_DIM), jnp.float32),
            pltpu.VMEM((2, _B, IRREPS_DIM), jnp.float32),
            pltpu.VMEM_SHARED((NUM_PAIRS, IRREPS_DIM), jnp.float32),
            pltpu.SemaphoreType.DMA,
            pltpu.SemaphoreType.DMA,
            pltpu.SemaphoreType.DMA,
            pltpu.SemaphoreType.DMA,
            pltpu.SemaphoreType.DMA,
            pltpu.SemaphoreType.DMA,
            pltpu.SemaphoreType.DMA,
            pltpu.SemaphoreType.DMA,
        ],
    )
    return call(edge_features, etype_flat, table_c)


def kernel(edge_features, edge_index, edge_type, scales, shifts):
    table_c = jnp.concatenate(
        [scales, shifts, jnp.zeros((NUM_PAIRS, NUM_SCALAR), jnp.float32)],
        axis=1,
    )
    etype_flat = edge_type.reshape(-1)
    return _run(edge_features, etype_flat, table_c)


# speedup vs baseline: 2.2680x; 2.2680x over previous
import functools

import numpy as np
import jax
import jax.numpy as jnp
from jax import lax
from jax.experimental import pallas as pl
from jax.experimental.pallas import tpu as pltpu
from jax.experimental.pallas import tpu_sc as plsc

NUM_PAIRS = 4096
IRREPS_DIM = 128
NUM_IRREPS = 64
NUM_SCALAR = 32
N_EDGES = 320000

_SHIFT_OFF = NUM_IRREPS

_INFO = plsc.get_sparse_core_info()
_NC, _NS, _L = _INFO.num_cores, _INFO.num_subcores, _INFO.num_lanes
_NW = _NC * _NS
_PER_W = N_EDGES // _NW
_B = 80
_CHUNKS = _PER_W // _B
_NVREG = IRREPS_DIM // _L


def _sc_body(feat_hbm, etype_hbm, table_hbm, out_hbm,
             idx_v, x_v, t_v, tab_sh,
             sem_tab, sem_idx, sem_t0, sem_t1, sem_t2,
             sem_x0, sem_x1, sem_x2, sem_o0, sem_o1, sem_o2):
    sid = lax.axis_index("s")
    wid = sid * _NC + lax.axis_index("c")
    w_base = wid * _PER_W

    @pl.when(sid == 0)
    def _():
        pltpu.async_copy(table_hbm, tab_sh, sem_tab).wait()

    pltpu.async_copy(etype_hbm.at[pl.ds(w_base, _PER_W)], idx_v,
                     sem_idx).wait()
    plsc.subcore_barrier()

    lane = lax.iota(jnp.int32, _L)
    lidx = []
    for j in range(2, _NVREG):
        col = ((16 * j - 32 + lane) * 10923) >> 15
        lidx.append(col - (0 if j <= 4 else 16))
    dnums = lax.GatherDimensionNumbers(
        offset_dims=(), collapsed_slice_dims=(0,), start_index_map=(0,))

    def perm(v, i):
        return lax.gather(v, i[:, None], dnums, (1,),
                          mode=lax.GatherScatterMode.PROMISE_IN_BOUNDS)

    sems_t = (sem_t0, sem_t1, sem_t2)
    sems_x = (sem_x0, sem_x1, sem_x2)
    sems_o = (sem_o0, sem_o1, sem_o2)

    def _in_args(g, slot):
        base = w_base + g * _B
        return ((tab_sh.at[idx_v.at[pl.ds(g * _B, _B)]], t_v.at[slot],
                 sems_t[slot]),
                (feat_hbm.at[pl.ds(base, _B)], x_v.at[slot], sems_x[slot]))

    def issue_in(g, slot):
        for a in _in_args(g, slot):
            pltpu.async_copy(*a)

    def wait_in(g, slot):
        for a in _in_args(g, slot):
            pltpu.make_async_copy(*a).wait()

    def _out_args(g, slot):
        base = w_base + g * _B
        return (x_v.at[slot], out_hbm.at[pl.ds(base, _B)], sems_o[slot])

    def issue_out(g, slot):
        pltpu.async_copy(*_out_args(g, slot))

    def wait_out(g, slot):
        pltpu.make_async_copy(*_out_args(g, slot)).wait()

    def compute(slot):
        def edge(e, c2):
            for j in range(2):
                sl = pl.ds(j * _L, _L)
                hsl = pl.ds(_SHIFT_OFF + j * _L, _L)
                x_v[slot, e, sl] = (x_v[slot, e, sl] * t_v[slot, e, sl]
                                    + t_v[slot, e, hsl])
            t2 = t_v[slot, e, pl.ds(2 * _L, _L)]
            t3 = t_v[slot, e, pl.ds(3 * _L, _L)]
            for j in range(2, _NVREG):
                sl = pl.ds(j * _L, _L)
                s = perm(t2 if j <= 4 else t3, lidx[j - 2])
                x_v[slot, e, sl] = x_v[slot, e, sl] * s
            return c2

        lax.fori_loop(0, _B, edge, 0, unroll=False)

    def step(g, slot):
        wait_in(g, slot)
        compute(slot)
        issue_out(g, slot)
        wait_out(g - 1, (g - 1) % 3)
        issue_in(g + 2, (g + 2) % 3)

    issue_in(0, 0)
    issue_in(1, 1)
    issue_in(2, 2)

    wait_in(0, 0)
    compute(0)
    issue_out(0, 0)

    def main(gg, carry):
        for b in (0, 1, 2):
            g_dyn = 3 * gg + 1 + b
            slot = (1 + b) % 3
            wait_in(g_dyn, slot)
            compute(slot)
            issue_out(g_dyn, slot)
            wait_out(g_dyn - 1, b % 3)
            issue_in(g_dyn + 2, b % 3)
        return carry

    _M = (_CHUNKS - 5) // 3
    lax.fori_loop(0, _M, main, 0, unroll=False)

    for g in range(3 * _M + 1, _CHUNKS - 2):
        step(g, g % 3)
    g = _CHUNKS - 2
    wait_in(g, g % 3)
    compute(g % 3)
    issue_out(g, g % 3)
    wait_out(g - 1, (g - 1) % 3)
    g = _CHUNKS - 1
    wait_in(g, g % 3)
    compute(g % 3)
    issue_out(g, g % 3)
    wait_out(g - 1, (g - 1) % 3)
    wait_out(g, g % 3)


@functools.partial(jax.jit, static_argnames=())
def _run(edge_features, etype_flat, table_c):
    mesh = plsc.VectorSubcoreMesh(core_axis_name="c", subcore_axis_name="s")
    call = pl.kernel(
        _sc_body,
        mesh=mesh,
        out_type=jax.ShapeDtypeStruct((N_EDGES, IRREPS_DIM), jnp.float32),
        scratch_types=[
            pltpu.VMEM((_PER_W,), jnp.int32),
            pltpu.VMEM((3, _B, IRREPS_DIM), jnp.float32),
            pltpu.VMEM((3, _B, IRREPS_DIM), jnp.float32),
            pltpu.VMEM_SHARED((NUM_PAIRS, IRREPS_DIM), jnp.float32),
            pltpu.SemaphoreType.DMA,
            pltpu.SemaphoreType.DMA,
            pltpu.SemaphoreType.DMA,
            pltpu.SemaphoreType.DMA,
            pltpu.SemaphoreType.DMA,
            pltpu.SemaphoreType.DMA,
            pltpu.SemaphoreType.DMA,
            pltpu.SemaphoreType.DMA,
            pltpu.SemaphoreType.DMA,
            pltpu.SemaphoreType.DMA,
            pltpu.SemaphoreType.DMA,
        ],
    )
    return call(edge_features, etype_flat, table_c)


def kernel(edge_features, edge_index, edge_type, scales, shifts):
    table_c = jnp.concatenate(
        [scales, shifts, jnp.zeros((NUM_PAIRS, NUM_SCALAR), jnp.float32)],
        axis=1,
    )
    etype_flat = edge_type.reshape(-1)
    return _run(edge_features, etype_flat, table_c)


# speedup vs baseline: 2.5834x; 1.1391x over previous
import functools

import numpy as np
import jax
import jax.numpy as jnp
from jax import lax
from jax.experimental import pallas as pl
from jax.experimental.pallas import tpu as pltpu
from jax.experimental.pallas import tpu_sc as plsc

NUM_PAIRS = 4096
IRREPS_DIM = 128
NUM_IRREPS = 64
NUM_SCALAR = 32
N_EDGES = 320000

_SHIFT_OFF = NUM_IRREPS

_INFO = plsc.get_sparse_core_info()
_NC, _NS, _L = _INFO.num_cores, _INFO.num_subcores, _INFO.num_lanes
_NW = _NC * _NS
_PER_W = N_EDGES // _NW
_B = 80
_CHUNKS = _PER_W // _B
_NBUF = 4
_NVREG = IRREPS_DIM // _L


def _sc_body(feat_hbm, etype_hbm, table_hbm, out_hbm,
             idx_v, x_v, t_v, tab_sh, sem_tab, sem_idx, *sems):
    sid = lax.axis_index("s")
    wid = sid * _NC + lax.axis_index("c")
    w_base = wid * _PER_W

    @pl.when(sid == 0)
    def _():
        pltpu.async_copy(table_hbm, tab_sh, sem_tab).wait()

    pltpu.async_copy(etype_hbm.at[pl.ds(w_base, _PER_W)], idx_v,
                     sem_idx).wait()
    plsc.subcore_barrier()

    lane = lax.iota(jnp.int32, _L)
    lidx = []
    for j in range(2, _NVREG):
        col = ((16 * j - 32 + lane) * 10923) >> 15
        lidx.append(col - (0 if j <= 4 else 16))
    dnums = lax.GatherDimensionNumbers(
        offset_dims=(), collapsed_slice_dims=(0,), start_index_map=(0,))

    def perm(v, i):
        return lax.gather(v, i[:, None], dnums, (1,),
                          mode=lax.GatherScatterMode.PROMISE_IN_BOUNDS)

    sems_t = sems[:_NBUF]
    sems_x = sems[_NBUF:2 * _NBUF]
    sems_o = sems[2 * _NBUF:3 * _NBUF]

    def _in_args(g, slot):
        base = w_base + g * _B
        return ((tab_sh.at[idx_v.at[pl.ds(g * _B, _B)]], t_v.at[slot],
                 sems_t[slot]),
                (feat_hbm.at[pl.ds(base, _B)], x_v.at[slot], sems_x[slot]))

    def issue_in(g, slot):
        for a in _in_args(g, slot):
            pltpu.async_copy(*a)

    def wait_in(g, slot):
        for a in _in_args(g, slot):
            pltpu.make_async_copy(*a).wait()

    def _out_args(g, slot):
        base = w_base + g * _B
        return (x_v.at[slot], out_hbm.at[pl.ds(base, _B)], sems_o[slot])

    def issue_out(g, slot):
        pltpu.async_copy(*_out_args(g, slot))

    def wait_out(g, slot):
        pltpu.make_async_copy(*_out_args(g, slot)).wait()

    def compute(slot):
        def edge(e, c2):
            for j in range(2):
                sl = pl.ds(j * _L, _L)
                hsl = pl.ds(_SHIFT_OFF + j * _L, _L)
                x_v[slot, e, sl] = (x_v[slot, e, sl] * t_v[slot, e, sl]
                                    + t_v[slot, e, hsl])
            t2 = t_v[slot, e, pl.ds(2 * _L, _L)]
            t3 = t_v[slot, e, pl.ds(3 * _L, _L)]
            for j in range(2, _NVREG):
                sl = pl.ds(j * _L, _L)
                s = perm(t2 if j <= 4 else t3, lidx[j - 2])
                x_v[slot, e, sl] = x_v[slot, e, sl] * s
            return c2

        lax.fori_loop(0, _B, edge, 0, unroll=False)

    def step(g, slot):
        wait_in(g, slot)
        compute(slot)
        issue_out(g, slot)
        wait_out(g - 1, (g - 1) % _NBUF)
        issue_in(g + _NBUF - 1, (g - 1) % _NBUF)

    for k in range(_NBUF):
        issue_in(k, k)

    wait_in(0, 0)
    compute(0)
    issue_out(0, 0)

    def main(gg, carry):
        for b in range(_NBUF):
            g_dyn = _NBUF * gg + 1 + b
            slot = (1 + b) % _NBUF
            wait_in(g_dyn, slot)
            compute(slot)
            issue_out(g_dyn, slot)
            wait_out(g_dyn - 1, b % _NBUF)
            issue_in(g_dyn + _NBUF - 1, b % _NBUF)
        return carry

    _M = (_CHUNKS - _NBUF) // _NBUF
    lax.fori_loop(0, _M, main, 0, unroll=False)

    for g in range(_NBUF * _M + 1, _CHUNKS - _NBUF + 1):
        step(g, g % _NBUF)
    for g in range(_CHUNKS - _NBUF + 1, _CHUNKS):
        wait_in(g, g % _NBUF)
        compute(g % _NBUF)
        issue_out(g, g % _NBUF)
        wait_out(g - 1, (g - 1) % _NBUF)
    wait_out(_CHUNKS - 1, (_CHUNKS - 1) % _NBUF)


@functools.partial(jax.jit, static_argnames=())
def _run(edge_features, etype_flat, table_c):
    mesh = plsc.VectorSubcoreMesh(core_axis_name="c", subcore_axis_name="s")
    call = pl.kernel(
        _sc_body,
        mesh=mesh,
        out_type=jax.ShapeDtypeStruct((N_EDGES, IRREPS_DIM), jnp.float32),
        scratch_types=[
            pltpu.VMEM((_PER_W,), jnp.int32),
            pltpu.VMEM((_NBUF, _B, IRREPS_DIM), jnp.float32),
            pltpu.VMEM((_NBUF, _B, IRREPS_DIM), jnp.float32),
            pltpu.VMEM_SHARED((NUM_PAIRS, IRREPS_DIM), jnp.float32),
            pltpu.SemaphoreType.DMA,
            pltpu.SemaphoreType.DMA,
        ] + [pltpu.SemaphoreType.DMA] * (3 * _NBUF) + [
        ],
    )
    return call(edge_features, etype_flat, table_c)


def kernel(edge_features, edge_index, edge_type, scales, shifts):
    table_c = jnp.concatenate(
        [scales, shifts, jnp.zeros((NUM_PAIRS, NUM_SCALAR), jnp.float32)],
        axis=1,
    )
    etype_flat = edge_type.reshape(-1)
    return _run(edge_features, etype_flat, table_c)
